# baseline (speedup 1.0000x reference)
import functools

import jax
import jax.numpy as jnp
from jax import lax
from jax.experimental import pallas as pl
from jax.experimental.pallas import tpu as pltpu
from jax.experimental.pallas import tpu_sc as plsc

N_NODES = 10000
N_EDGES = 320000

NC = 2
NS = 16
LANES = 16

NW = NC * NS
CHUNK = 128
NCHUNK = 79
EPW = NCHUNK * CHUNK
E_PAD = NW * EPW
DUMMY = N_NODES
ZROWS = 125
RPT = N_NODES // NS
RPT_RD = 624
DCOL = 64


def _sc_segment_sum(parts, src, dst):
    p_total = len(parts)
    mesh = plsc.VectorSubcoreMesh(core_axis_name="c", subcore_axis_name="s")

    @functools.partial(
        pl.kernel,
        out_type=jax.ShapeDtypeStruct((p_total, NC, N_NODES, DCOL),
                                      jnp.float32),
        mesh=mesh,
        scratch_types=[
            pltpu.VMEM((NCHUNK, CHUNK), jnp.int32),
            pltpu.VMEM((NCHUNK, CHUNK), jnp.int32),
            pltpu.VMEM((CHUNK, DCOL), jnp.float32),
            pltpu.VMEM((CHUNK, DCOL), jnp.float32),
            pltpu.VMEM((ZROWS, DCOL), jnp.float32),
            pltpu.VMEM_SHARED((N_NODES + 8, DCOL), jnp.float32),
            pltpu.SemaphoreType.DMA,
            pltpu.SemaphoreType.DMA,
        ],
        compiler_params=pltpu.CompilerParams(use_tc_tiling_on_sc=False),
    )
    def kern(*refs):
        x_hbms = refs[:p_total]
        (src_hbm, dst_hbm, out_hbm, src_v, dst_v, buf_a, buf_b, z_v, acc_sh,
         sem_a, sem_b) = refs[p_total:]
        cid = lax.axis_index("c")
        sid = lax.axis_index("s")
        wid = cid * NS + sid

        zv = jnp.zeros((LANES,), jnp.float32)

        @pl.loop(0, ZROWS)
        def _(r):
            for k in range(DCOL // LANES):
                z_v[r, pl.ds(k * LANES, LANES)] = zv

        pltpu.sync_copy(src_hbm.at[wid], src_v)
        pltpu.sync_copy(dst_hbm.at[wid], dst_v)

        for p in range(p_total):
            for j in range(RPT // ZROWS):
                pltpu.sync_copy(
                    z_v, acc_sh.at[pl.ds(sid * RPT + j * ZROWS, ZROWS)])
            plsc.subcore_barrier()

            x_hbm = x_hbms[p]

            def wait_gather(buf, sem):
                pltpu.make_async_copy(x_hbm.at[pl.ds(0, CHUNK)], buf,
                                      sem).wait()

            pltpu.async_copy(x_hbm.at[src_v.at[0]], buf_a, sem_a)

            @pl.loop(0, (NCHUNK - 1) // 2)
            def _(i):
                c1 = 2 * i + 1
                pltpu.async_copy(x_hbm.at[src_v.at[c1]], buf_b, sem_b)
                wait_gather(buf_a, sem_a)
                pltpu.sync_copy(buf_a, acc_sh.at[dst_v.at[2 * i]], add=True)
                pltpu.async_copy(x_hbm.at[src_v.at[c1 + 1]], buf_a, sem_a)
                wait_gather(buf_b, sem_b)
                pltpu.sync_copy(buf_b, acc_sh.at[dst_v.at[c1]], add=True)

            wait_gather(buf_a, sem_a)
            pltpu.sync_copy(buf_a, acc_sh.at[dst_v.at[NCHUNK - 1]], add=True)

            plsc.subcore_barrier()

            pltpu.sync_copy(acc_sh.at[pl.ds(sid * RPT_RD, RPT_RD)],
                            out_hbm.at[p, cid, pl.ds(sid * RPT_RD, RPT_RD)])

            @pl.when(sid == NS - 1)
            def _():
                tail = N_NODES - NS * RPT_RD
                pltpu.sync_copy(
                    acc_sh.at[pl.ds(NS * RPT_RD, tail)],
                    out_hbm.at[p, cid, pl.ds(NS * RPT_RD, tail)])

            if p + 1 < p_total:
                plsc.subcore_barrier()

    return kern(*parts, src, dst)


def _dense_layer(acc, x, w_rel_t, b_rel, w_root_t):
    n, d_in = x.shape
    p_total = acc.shape[0]
    d_out = w_rel_t.shape[2]
    blk = 1000

    def body(acc_ref, x_ref, wr_ref, b_ref, wt_ref, o_ref):
        y = jnp.dot(x_ref[...], wt_ref[...],
                    precision=lax.Precision.HIGHEST,
                    preferred_element_type=jnp.float32)
        for p in range(p_total):
            a = acc_ref[p, 0] + acc_ref[p, 1]
            y = y + jnp.dot(a, wr_ref[p],
                            precision=lax.Precision.HIGHEST,
                            preferred_element_type=jnp.float32)
        y = y + b_ref[...]
        o_ref[...] = jnp.where(y >= 0, y, 0.01 * y)

    return pl.pallas_call(
        body,
        grid=(n // blk,),
        in_specs=[
            pl.BlockSpec((p_total, NC, blk, DCOL), lambda i: (0, 0, i, 0)),
            pl.BlockSpec((blk, d_in), lambda i: (i, 0)),
            pl.BlockSpec((p_total, DCOL, d_out), lambda i: (0, 0, 0)),
            pl.BlockSpec((1, d_out), lambda i: (0, 0)),
            pl.BlockSpec((d_in, d_out), lambda i: (0, 0)),
        ],
        out_specs=pl.BlockSpec((blk, d_out), lambda i: (i, 0)),
        out_shape=jax.ShapeDtypeStruct((n, d_out), jnp.float32),
    )(acc, x, w_rel_t, b_rel, w_root_t)


def kernel(x, edge_index,
           W_rel_e0, b_rel_e0, W_root_e0,
           W_rel_e1, b_rel_e1, W_root_e1,
           W_rel_d0, b_rel_d0, W_root_d0,
           W_rel_d1, b_rel_d1, W_root_d1):
    pad = E_PAD - N_EDGES
    src = jnp.concatenate(
        [edge_index[0], jnp.zeros((pad,), jnp.int32)]).reshape(
            NW, NCHUNK, CHUNK)
    dst = jnp.concatenate(
        [edge_index[1], jnp.full((pad,), DUMMY, jnp.int32)]).reshape(
            NW, NCHUNK, CHUNK)

    def layer(feat, w_rel, b_rel, w_root):
        d = feat.shape[1]
        parts = tuple(feat[:, p * DCOL:(p + 1) * DCOL]
                      for p in range(d // DCOL))
        acc = _sc_segment_sum(parts, src, dst)
        wr_t = w_rel.T.reshape(len(parts), DCOL, -1)
        return _dense_layer(acc, feat, wr_t, b_rel.reshape(1, -1), w_root.T)

    h = layer(x, W_rel_e0, b_rel_e0, W_root_e0)
    emb = layer(h, W_rel_e1, b_rel_e1, W_root_e1)
    h2 = layer(emb, W_rel_d0, b_rel_d0, W_root_d0)
    recon = layer(h2, W_rel_d1, b_rel_d1, W_root_d1)
    return (recon, emb)


# speedup vs baseline: 1.8826x; 1.8826x over previous
import functools

import jax
import jax.numpy as jnp
from jax import lax
from jax.experimental import pallas as pl
from jax.experimental.pallas import tpu as pltpu
from jax.experimental.pallas import tpu_sc as plsc

N_NODES = 10000
N_EDGES = 320000

NC = 2
NS = 16
LANES = 16

NW = NC * NS
CHUNK = 80
EPW = N_EDGES // NW
NCHUNK = EPW // CHUNK
ZROWS = 125
RPT = N_NODES // NS
RPT_RD = 624
DCOL = 64


def _sc_segment_sum(parts, src, dst):
    p_total = len(parts)
    mesh = plsc.VectorSubcoreMesh(core_axis_name="c", subcore_axis_name="s")

    @functools.partial(
        pl.kernel,
        out_type=jax.ShapeDtypeStruct((p_total, NC, N_NODES, DCOL),
                                      jnp.float32),
        mesh=mesh,
        scratch_types=[
            pltpu.VMEM((NCHUNK, CHUNK), jnp.int32),
            pltpu.VMEM((NCHUNK, CHUNK), jnp.int32),
            pltpu.VMEM((CHUNK, DCOL), jnp.float32),
            pltpu.VMEM((CHUNK, DCOL), jnp.float32),
            pltpu.VMEM((CHUNK, DCOL), jnp.float32),
            pltpu.VMEM((CHUNK, DCOL), jnp.float32),
            pltpu.VMEM((ZROWS, DCOL), jnp.float32),
            pltpu.VMEM_SHARED((N_NODES + 8, DCOL), jnp.float32),
            pltpu.SemaphoreType.DMA,
            pltpu.SemaphoreType.DMA,
            pltpu.SemaphoreType.DMA,
            pltpu.SemaphoreType.DMA,
            pltpu.SemaphoreType.DMA,
            pltpu.SemaphoreType.DMA,
            pltpu.SemaphoreType.DMA,
            pltpu.SemaphoreType.DMA,
        ],
        compiler_params=pltpu.CompilerParams(use_tc_tiling_on_sc=False),
    )
    def kern(*refs):
        x_hbms = refs[:p_total]
        (src_hbm, dst_hbm, out_hbm, src_v, dst_v) = refs[p_total:p_total + 5]
        bufs = refs[p_total + 5:p_total + 9]
        z_v, acc_sh = refs[p_total + 9:p_total + 11]
        sem_g = refs[p_total + 11:p_total + 15]
        sem_a = refs[p_total + 15:p_total + 19]
        cid = lax.axis_index("c")
        sid = lax.axis_index("s")
        wid = cid * NS + sid

        zv = jnp.zeros((LANES,), jnp.float32)

        @pl.loop(0, ZROWS)
        def _(r):
            for k in range(DCOL // LANES):
                z_v[r, pl.ds(k * LANES, LANES)] = zv

        pltpu.sync_copy(src_hbm.at[wid], src_v)
        pltpu.sync_copy(dst_hbm.at[wid], dst_v)

        for p in range(p_total):
            for j in range(RPT // ZROWS):
                pltpu.sync_copy(
                    z_v, acc_sh.at[pl.ds(sid * RPT + j * ZROWS, ZROWS)])
            plsc.subcore_barrier()

            x_hbm = x_hbms[p]

            def wait_gather(b):
                pltpu.make_async_copy(x_hbm.at[pl.ds(0, CHUNK)], bufs[b],
                                      sem_g[b]).wait()

            def wait_add(b, c):
                pltpu.make_async_copy(bufs[b], acc_sh.at[dst_v.at[c]],
                                      sem_a[b]).wait()

            for b in range(4):
                pltpu.async_copy(x_hbm.at[src_v.at[b]], bufs[b], sem_g[b])

            @pl.loop(0, (NCHUNK - 1) // 4)
            def _(i):
                for b in range(4):
                    c = 4 * i + b
                    wait_gather(b)
                    pltpu.async_copy(bufs[b], acc_sh.at[dst_v.at[c]],
                                     sem_a[b], add=True)
                for b in range(4):
                    c = 4 * i + b
                    cn = c + 4
                    wait_add(b, c)

                    @pl.when(cn < NCHUNK)
                    def _():
                        pltpu.async_copy(x_hbm.at[src_v.at[cn]], bufs[b],
                                         sem_g[b])

            wait_gather(0)
            pltpu.sync_copy(bufs[0], acc_sh.at[dst_v.at[NCHUNK - 1]],
                            add=True)

            plsc.subcore_barrier()

            pltpu.sync_copy(acc_sh.at[pl.ds(sid * RPT_RD, RPT_RD)],
                            out_hbm.at[p, cid, pl.ds(sid * RPT_RD, RPT_RD)])

            @pl.when(sid == NS - 1)
            def _():
                tail = N_NODES - NS * RPT_RD
                pltpu.sync_copy(
                    acc_sh.at[pl.ds(NS * RPT_RD, tail)],
                    out_hbm.at[p, cid, pl.ds(NS * RPT_RD, tail)])

            if p + 1 < p_total:
                plsc.subcore_barrier()

    return kern(*parts, src, dst)


def _dense_layer(acc, x, w_rel_t, b_rel, w_root_t):
    n, d_in = x.shape
    p_total = acc.shape[0]
    d_out = w_rel_t.shape[2]
    blk = 1000

    def body(acc_ref, x_ref, wr_ref, b_ref, wt_ref, o_ref):
        y = jnp.dot(x_ref[...], wt_ref[...],
                    precision=lax.Precision.HIGHEST,
                    preferred_element_type=jnp.float32)
        for p in range(p_total):
            a = acc_ref[p, 0] + acc_ref[p, 1]
            y = y + jnp.dot(a, wr_ref[p],
                            precision=lax.Precision.HIGHEST,
                            preferred_element_type=jnp.float32)
        y = y + b_ref[...]
        o_ref[...] = jnp.where(y >= 0, y, 0.01 * y)

    return pl.pallas_call(
        body,
        grid=(n // blk,),
        in_specs=[
            pl.BlockSpec((p_total, NC, blk, DCOL), lambda i: (0, 0, i, 0)),
            pl.BlockSpec((blk, d_in), lambda i: (i, 0)),
            pl.BlockSpec((p_total, DCOL, d_out), lambda i: (0, 0, 0)),
            pl.BlockSpec((1, d_out), lambda i: (0, 0)),
            pl.BlockSpec((d_in, d_out), lambda i: (0, 0)),
        ],
        out_specs=pl.BlockSpec((blk, d_out), lambda i: (i, 0)),
        out_shape=jax.ShapeDtypeStruct((n, d_out), jnp.float32),
    )(acc, x, w_rel_t, b_rel, w_root_t)


def kernel(x, edge_index,
           W_rel_e0, b_rel_e0, W_root_e0,
           W_rel_e1, b_rel_e1, W_root_e1,
           W_rel_d0, b_rel_d0, W_root_d0,
           W_rel_d1, b_rel_d1, W_root_d1):
    src = edge_index[0].reshape(NW, NCHUNK, CHUNK)
    dst = edge_index[1].reshape(NW, NCHUNK, CHUNK)

    def layer(feat, w_rel, b_rel, w_root):
        d = feat.shape[1]
        parts = tuple(feat[:, p * DCOL:(p + 1) * DCOL]
                      for p in range(d // DCOL))
        acc = _sc_segment_sum(parts, src, dst)
        wr_t = w_rel.T.reshape(len(parts), DCOL, -1)
        return _dense_layer(acc, feat, wr_t, b_rel.reshape(1, -1), w_root.T)

    h = layer(x, W_rel_e0, b_rel_e0, W_root_e0)
    emb = layer(h, W_rel_e1, b_rel_e1, W_root_e1)
    h2 = layer(emb, W_rel_d0, b_rel_d0, W_root_d0)
    recon = layer(h2, W_rel_d1, b_rel_d1, W_root_d1)
    return (recon, emb)


# speedup vs baseline: 1.9640x; 1.0433x over previous
import functools

import jax
import jax.numpy as jnp
from jax import lax
from jax.experimental import pallas as pl
from jax.experimental.pallas import tpu as pltpu
from jax.experimental.pallas import tpu_sc as plsc

N_NODES = 10000
N_EDGES = 320000

NC = 2
NS = 16
LANES = 16

NW = NC * NS
CHUNK = 80
EPR = N_EDGES // NW
NCHUNK = EPR // CHUNK
ZROWS = 125
RPT = N_NODES // NS
RPT_RD = 624
DCOL = 64
NBUF = 4


def _sc_segment_sum(parts, src, dst):
    p_total = len(parts)
    assert p_total in (1, 2)
    rpt_ranges = 2 if p_total == 2 else 1
    nch = NCHUNK * rpt_ranges
    mesh = plsc.VectorSubcoreMesh(core_axis_name="c", subcore_axis_name="s")

    @functools.partial(
        pl.kernel,
        out_type=jax.ShapeDtypeStruct((2, N_NODES, DCOL), jnp.float32),
        mesh=mesh,
        scratch_types=[
            pltpu.VMEM((nch, CHUNK), jnp.int32),
            pltpu.VMEM((nch, CHUNK), jnp.int32),
            pltpu.VMEM((CHUNK, DCOL), jnp.float32),
            pltpu.VMEM((CHUNK, DCOL), jnp.float32),
            pltpu.VMEM((CHUNK, DCOL), jnp.float32),
            pltpu.VMEM((CHUNK, DCOL), jnp.float32),
            pltpu.VMEM((ZROWS, DCOL), jnp.float32),
            pltpu.VMEM_SHARED((N_NODES + 8, DCOL), jnp.float32),
            pltpu.SemaphoreType.DMA,
            pltpu.SemaphoreType.DMA,
            pltpu.SemaphoreType.DMA,
            pltpu.SemaphoreType.DMA,
            pltpu.SemaphoreType.DMA,
            pltpu.SemaphoreType.DMA,
            pltpu.SemaphoreType.DMA,
            pltpu.SemaphoreType.DMA,
        ],
        compiler_params=pltpu.CompilerParams(use_tc_tiling_on_sc=False),
    )
    def kern(*refs):
        x_hbms = refs[:p_total]
        (src_hbm, dst_hbm, out_hbm, src_v, dst_v) = refs[p_total:p_total + 5]
        bufs = refs[p_total + 5:p_total + 9]
        z_v, acc_sh = refs[p_total + 9:p_total + 11]
        sem_g = refs[p_total + 11:p_total + 15]
        sem_a = refs[p_total + 15:p_total + 19]
        cid = lax.axis_index("c")
        sid = lax.axis_index("s")

        zv = jnp.zeros((LANES,), jnp.float32)

        @pl.loop(0, ZROWS)
        def _(r):
            for k in range(DCOL // LANES):
                z_v[r, pl.ds(k * LANES, LANES)] = zv

        if p_total == 2:
            for r in range(2):
                pltpu.sync_copy(src_hbm.at[sid + r * NS],
                                src_v.at[pl.ds(r * NCHUNK, NCHUNK)])
                pltpu.sync_copy(dst_hbm.at[sid + r * NS],
                                dst_v.at[pl.ds(r * NCHUNK, NCHUNK)])
        else:
            wid = cid * NS + sid
            pltpu.sync_copy(src_hbm.at[wid], src_v)
            pltpu.sync_copy(dst_hbm.at[wid], dst_v)

        for j in range(RPT // ZROWS):
            pltpu.sync_copy(z_v, acc_sh.at[pl.ds(sid * RPT + j * ZROWS,
                                                 ZROWS)])
        plsc.subcore_barrier()

        def run_slab(x_hbm):
            def wait_gather(b):
                pltpu.make_async_copy(x_hbm.at[pl.ds(0, CHUNK)], bufs[b],
                                      sem_g[b]).wait()

            def wait_add(b, c):
                pltpu.make_async_copy(bufs[b], acc_sh.at[dst_v.at[c]],
                                      sem_a[b]).wait()

            for b in range(NBUF):
                pltpu.async_copy(x_hbm.at[src_v.at[b]], bufs[b], sem_g[b])

            nmain = (nch - 1) // NBUF * NBUF

            @pl.loop(0, nmain // NBUF)
            def _(i):
                for b in range(NBUF):
                    c = NBUF * i + b
                    wait_gather(b)
                    pltpu.async_copy(bufs[b], acc_sh.at[dst_v.at[c]],
                                     sem_a[b], add=True)
                for b in range(NBUF):
                    c = NBUF * i + b
                    cn = c + NBUF
                    wait_add(b, c)

                    @pl.when(cn < nch)
                    def _():
                        pltpu.async_copy(x_hbm.at[src_v.at[cn]], bufs[b],
                                         sem_g[b])

            for c in range(nmain, nch):
                b = c % NBUF
                wait_gather(b)
                pltpu.sync_copy(bufs[b], acc_sh.at[dst_v.at[c]], add=True)

        if p_total == 2:
            @pl.when(cid == 0)
            def _():
                run_slab(x_hbms[0])

            @pl.when(cid == 1)
            def _():
                run_slab(x_hbms[1])
        else:
            run_slab(x_hbms[0])

        plsc.subcore_barrier()

        pltpu.sync_copy(acc_sh.at[pl.ds(sid * RPT_RD, RPT_RD)],
                        out_hbm.at[cid, pl.ds(sid * RPT_RD, RPT_RD)])

        @pl.when(sid == NS - 1)
        def _():
            tail = N_NODES - NS * RPT_RD
            pltpu.sync_copy(
                acc_sh.at[pl.ds(NS * RPT_RD, tail)],
                out_hbm.at[cid, pl.ds(NS * RPT_RD, tail)])

    return kern(*parts, src, dst)


def _dense_layer(acc, xs, w_rel_t, b_rel, w_root_t, p_total, out_slabs):
    n = acc.shape[1]
    d_out = w_rel_t.shape[2]
    x_total = len(xs)
    blk = 1000

    def body(*refs):
        acc_ref = refs[0]
        x_refs = refs[1:1 + x_total]
        wr_ref, b_ref, wt_ref = refs[1 + x_total:4 + x_total]
        o_refs = refs[4 + x_total:]
        y = b_ref[...]
        if p_total == 2:
            for p in range(2):
                y = y + jnp.dot(acc_ref[p], wr_ref[p],
                                precision=lax.Precision.HIGHEST,
                                preferred_element_type=jnp.float32)
        else:
            a = acc_ref[0] + acc_ref[1]
            y = y + jnp.dot(a, wr_ref[0],
                            precision=lax.Precision.HIGHEST,
                            preferred_element_type=jnp.float32)
        for q in range(x_total):
            y = y + jnp.dot(x_refs[q][...], wt_ref[q],
                            precision=lax.Precision.HIGHEST,
                            preferred_element_type=jnp.float32)
        y = jnp.where(y >= 0, y, 0.01 * y)
        ow = d_out // out_slabs
        for s in range(out_slabs):
            o_refs[s][...] = y[:, s * ow:(s + 1) * ow]

    return pl.pallas_call(
        body,
        grid=(n // blk,),
        in_specs=[pl.BlockSpec((2, blk, DCOL), lambda i: (0, i, 0))]
        + [pl.BlockSpec((blk, DCOL), lambda i: (i, 0))] * x_total
        + [
            pl.BlockSpec((p_total, DCOL, d_out), lambda i: (0, 0, 0)),
            pl.BlockSpec((1, d_out), lambda i: (0, 0)),
            pl.BlockSpec((x_total, DCOL, d_out), lambda i: (0, 0, 0)),
        ],
        out_specs=[pl.BlockSpec((blk, d_out // out_slabs),
                                lambda i: (i, 0))] * out_slabs,
        out_shape=[jax.ShapeDtypeStruct((n, d_out // out_slabs),
                                        jnp.float32)] * out_slabs,
    )(acc, *xs, w_rel_t, b_rel, w_root_t)


def kernel(x, edge_index,
           W_rel_e0, b_rel_e0, W_root_e0,
           W_rel_e1, b_rel_e1, W_root_e1,
           W_rel_d0, b_rel_d0, W_root_d0,
           W_rel_d1, b_rel_d1, W_root_d1):
    src = edge_index[0].reshape(NW, NCHUNK, CHUNK)
    dst = edge_index[1].reshape(NW, NCHUNK, CHUNK)

    def layer(x_slabs, w_rel, b_rel, w_root, out_slabs):
        p_total = len(x_slabs)
        acc = _sc_segment_sum(x_slabs, src, dst)
        wr_t = w_rel.T.reshape(p_total, DCOL, -1)
        wt_t = w_root.T.reshape(p_total, DCOL, -1)
        outs = _dense_layer(acc, x_slabs, wr_t, b_rel.reshape(1, -1), wt_t,
                            p_total, out_slabs)
        return tuple(outs)

    x_slabs = (x[:, :DCOL], x[:, DCOL:])
    h = layer(x_slabs, W_rel_e0, b_rel_e0, W_root_e0, 2)
    emb_t = layer(h, W_rel_e1, b_rel_e1, W_root_e1, 1)
    h2 = layer(emb_t, W_rel_d0, b_rel_d0, W_root_d0, 2)
    (recon,) = layer(h2, W_rel_d1, b_rel_d1, W_root_d1, 1)
    return (recon, emb_t[0])


# speedup vs baseline: 2.0799x; 1.0590x over previous
import functools

import jax
import jax.numpy as jnp
from jax import lax
from jax.experimental import pallas as pl
from jax.experimental.pallas import tpu as pltpu
from jax.experimental.pallas import tpu_sc as plsc

N_NODES = 10000
N_EDGES = 320000

NC = 2
NS = 16
LANES = 16

NW = NC * NS
CHUNK = 80
EPR = N_EDGES // NW
NCHUNK = EPR // CHUNK
ZROWS = 125
RPT = N_NODES // NS
RPT_RD = 624
DCOL = 64
NBUF = 8


def _sc_segment_sum(parts, src, dst):
    p_total = len(parts)
    assert p_total in (1, 2)
    rpt_ranges = 2 if p_total == 2 else 1
    nch = NCHUNK * rpt_ranges
    mesh = plsc.VectorSubcoreMesh(core_axis_name="c", subcore_axis_name="s")

    @functools.partial(
        pl.kernel,
        out_type=jax.ShapeDtypeStruct((2, N_NODES, DCOL), jnp.float32),
        mesh=mesh,
        scratch_types=[
            pltpu.VMEM((nch, CHUNK), jnp.int32),
            pltpu.VMEM((nch, CHUNK), jnp.int32),
            *([pltpu.VMEM((CHUNK, DCOL), jnp.float32)] * NBUF),
            pltpu.VMEM((ZROWS, DCOL), jnp.float32),
            pltpu.VMEM_SHARED((N_NODES + 8, DCOL), jnp.float32),
            *([pltpu.SemaphoreType.DMA] * NBUF),
            *([pltpu.SemaphoreType.DMA] * NBUF),
        ],
        compiler_params=pltpu.CompilerParams(use_tc_tiling_on_sc=False),
    )
    def kern(*refs):
        x_hbms = refs[:p_total]
        (src_hbm, dst_hbm, out_hbm, src_v, dst_v) = refs[p_total:p_total + 5]
        k0 = p_total + 5
        bufs = refs[k0:k0 + NBUF]
        z_v, acc_sh = refs[k0 + NBUF:k0 + NBUF + 2]
        sem_g = refs[k0 + NBUF + 2:k0 + 2 * NBUF + 2]
        sem_a = refs[k0 + 2 * NBUF + 2:k0 + 3 * NBUF + 2]
        cid = lax.axis_index("c")
        sid = lax.axis_index("s")

        zv = jnp.zeros((LANES,), jnp.float32)

        @pl.loop(0, ZROWS)
        def _(r):
            for k in range(DCOL // LANES):
                z_v[r, pl.ds(k * LANES, LANES)] = zv

        if p_total == 2:
            for r in range(2):
                pltpu.sync_copy(src_hbm.at[sid + r * NS],
                                src_v.at[pl.ds(r * NCHUNK, NCHUNK)])
                pltpu.sync_copy(dst_hbm.at[sid + r * NS],
                                dst_v.at[pl.ds(r * NCHUNK, NCHUNK)])
        else:
            wid = cid * NS + sid
            pltpu.sync_copy(src_hbm.at[wid], src_v)
            pltpu.sync_copy(dst_hbm.at[wid], dst_v)

        for j in range(RPT // ZROWS):
            pltpu.sync_copy(z_v, acc_sh.at[pl.ds(sid * RPT + j * ZROWS,
                                                 ZROWS)])
        plsc.subcore_barrier()

        def run_slab(x_hbm):
            def wait_gather(b):
                pltpu.make_async_copy(x_hbm.at[pl.ds(0, CHUNK)], bufs[b],
                                      sem_g[b]).wait()

            def wait_add(b, c):
                pltpu.make_async_copy(bufs[b], acc_sh.at[dst_v.at[c]],
                                      sem_a[b]).wait()

            for b in range(NBUF):
                pltpu.async_copy(x_hbm.at[src_v.at[b]], bufs[b], sem_g[b])

            nmain = (nch - 1) // NBUF * NBUF

            @pl.loop(0, nmain // NBUF)
            def _(i):
                for b in range(NBUF):
                    c = NBUF * i + b
                    wait_gather(b)
                    pltpu.async_copy(bufs[b], acc_sh.at[dst_v.at[c]],
                                     sem_a[b], add=True)
                for b in range(NBUF):
                    c = NBUF * i + b
                    cn = c + NBUF
                    wait_add(b, c)

                    @pl.when(cn < nch)
                    def _():
                        pltpu.async_copy(x_hbm.at[src_v.at[cn]], bufs[b],
                                         sem_g[b])

            for c in range(nmain, nch):
                b = c % NBUF
                wait_gather(b)
                pltpu.sync_copy(bufs[b], acc_sh.at[dst_v.at[c]], add=True)

        if p_total == 2:
            @pl.when(cid == 0)
            def _():
                run_slab(x_hbms[0])

            @pl.when(cid == 1)
            def _():
                run_slab(x_hbms[1])
        else:
            run_slab(x_hbms[0])

        plsc.subcore_barrier()

        pltpu.sync_copy(acc_sh.at[pl.ds(sid * RPT_RD, RPT_RD)],
                        out_hbm.at[cid, pl.ds(sid * RPT_RD, RPT_RD)])

        @pl.when(sid == NS - 1)
        def _():
            tail = N_NODES - NS * RPT_RD
            pltpu.sync_copy(
                acc_sh.at[pl.ds(NS * RPT_RD, tail)],
                out_hbm.at[cid, pl.ds(NS * RPT_RD, tail)])

    return kern(*parts, src, dst)


def _dense_layer(acc, xs, w_rel_t, b_rel, w_root_t, p_total, out_slabs):
    n = acc.shape[1]
    d_out = w_rel_t.shape[2]
    x_total = len(xs)
    blk = 1000

    def body(*refs):
        acc_ref = refs[0]
        x_refs = refs[1:1 + x_total]
        wr_ref, b_ref, wt_ref = refs[1 + x_total:4 + x_total]
        o_refs = refs[4 + x_total:]
        y = b_ref[...]
        if p_total == 2:
            for p in range(2):
                y = y + jnp.dot(acc_ref[p], wr_ref[p],
                                precision=lax.Precision.HIGHEST,
                                preferred_element_type=jnp.float32)
        else:
            a = acc_ref[0] + acc_ref[1]
            y = y + jnp.dot(a, wr_ref[0],
                            precision=lax.Precision.HIGHEST,
                            preferred_element_type=jnp.float32)
        for q in range(x_total):
            y = y + jnp.dot(x_refs[q][...], wt_ref[q],
                            precision=lax.Precision.HIGHEST,
                            preferred_element_type=jnp.float32)
        y = jnp.where(y >= 0, y, 0.01 * y)
        ow = d_out // out_slabs
        for s in range(out_slabs):
            o_refs[s][...] = y[:, s * ow:(s + 1) * ow]

    return pl.pallas_call(
        body,
        grid=(n // blk,),
        in_specs=[pl.BlockSpec((2, blk, DCOL), lambda i: (0, i, 0))]
        + [pl.BlockSpec((blk, DCOL), lambda i: (i, 0))] * x_total
        + [
            pl.BlockSpec((p_total, DCOL, d_out), lambda i: (0, 0, 0)),
            pl.BlockSpec((1, d_out), lambda i: (0, 0)),
            pl.BlockSpec((x_total, DCOL, d_out), lambda i: (0, 0, 0)),
        ],
        out_specs=[pl.BlockSpec((blk, d_out // out_slabs),
                                lambda i: (i, 0))] * out_slabs,
        out_shape=[jax.ShapeDtypeStruct((n, d_out // out_slabs),
                                        jnp.float32)] * out_slabs,
    )(acc, *xs, w_rel_t, b_rel, w_root_t)


def kernel(x, edge_index,
           W_rel_e0, b_rel_e0, W_root_e0,
           W_rel_e1, b_rel_e1, W_root_e1,
           W_rel_d0, b_rel_d0, W_root_d0,
           W_rel_d1, b_rel_d1, W_root_d1):
    src = edge_index[0].reshape(NW, NCHUNK, CHUNK)
    dst = edge_index[1].reshape(NW, NCHUNK, CHUNK)

    def layer(x_slabs, w_rel, b_rel, w_root, out_slabs):
        p_total = len(x_slabs)
        acc = _sc_segment_sum(x_slabs, src, dst)
        wr_t = w_rel.T.reshape(p_total, DCOL, -1)
        wt_t = w_root.T.reshape(p_total, DCOL, -1)
        outs = _dense_layer(acc, x_slabs, wr_t, b_rel.reshape(1, -1), wt_t,
                            p_total, out_slabs)
        return tuple(outs)

    x_slabs = (x[:, :DCOL], x[:, DCOL:])
    h = layer(x_slabs, W_rel_e0, b_rel_e0, W_root_e0, 2)
    emb_t = layer(h, W_rel_e1, b_rel_e1, W_root_e1, 1)
    h2 = layer(emb_t, W_rel_d0, b_rel_d0, W_root_d0, 2)
    (recon,) = layer(h2, W_rel_d1, b_rel_d1, W_root_d1, 1)
    return (recon, emb_t[0])


# speedup vs baseline: 2.1281x; 1.0232x over previous
import functools

import jax
import jax.numpy as jnp
from jax import lax
from jax.experimental import pallas as pl
from jax.experimental.pallas import tpu as pltpu
from jax.experimental.pallas import tpu_sc as plsc

N_NODES = 10000
N_EDGES = 320000

NC = 2
NS = 16
LANES = 16

NW = NC * NS
CHUNK = 80
EPR = N_EDGES // NW
NCHUNK = EPR // CHUNK
ZROWS = 125
RPT = N_NODES // NS
RPT_RD = 624
DCOL = 64
NBUF = 8


def _sc_segment_sum(parts, src, dst):
    p_total = len(parts)
    assert p_total in (1, 2)
    rpt_ranges = 2 if p_total == 2 else 1
    nch = NCHUNK * rpt_ranges
    mesh = plsc.VectorSubcoreMesh(core_axis_name="c", subcore_axis_name="s")

    @functools.partial(
        pl.kernel,
        out_type=jax.ShapeDtypeStruct((2, N_NODES, DCOL), jnp.float32),
        mesh=mesh,
        scratch_types=[
            pltpu.VMEM((nch, CHUNK), jnp.int32),
            pltpu.VMEM((nch, CHUNK), jnp.int32),
            *([pltpu.VMEM((CHUNK, DCOL), jnp.float32)] * NBUF),
            pltpu.VMEM((ZROWS, DCOL), jnp.float32),
            pltpu.VMEM_SHARED((N_NODES + 8, DCOL), jnp.float32),
            *([pltpu.SemaphoreType.DMA] * NBUF),
            *([pltpu.SemaphoreType.DMA] * NBUF),
        ],
        compiler_params=pltpu.CompilerParams(use_tc_tiling_on_sc=False),
    )
    def kern(*refs):
        x_hbms = refs[:p_total]
        (src_hbm, dst_hbm, out_hbm, src_v, dst_v) = refs[p_total:p_total + 5]
        k0 = p_total + 5
        bufs = refs[k0:k0 + NBUF]
        z_v, acc_sh = refs[k0 + NBUF:k0 + NBUF + 2]
        sem_g = refs[k0 + NBUF + 2:k0 + 2 * NBUF + 2]
        sem_a = refs[k0 + 2 * NBUF + 2:k0 + 3 * NBUF + 2]
        cid = lax.axis_index("c")
        sid = lax.axis_index("s")

        zv = jnp.zeros((LANES,), jnp.float32)

        @pl.loop(0, ZROWS)
        def _(r):
            for k in range(DCOL // LANES):
                z_v[r, pl.ds(k * LANES, LANES)] = zv

        if p_total == 2:
            for r in range(2):
                pltpu.async_copy(src_hbm.at[sid + r * NS],
                                 src_v.at[pl.ds(r * NCHUNK, NCHUNK)],
                                 sem_g[0])
                pltpu.async_copy(dst_hbm.at[sid + r * NS],
                                 dst_v.at[pl.ds(r * NCHUNK, NCHUNK)],
                                 sem_g[0])
        else:
            wid = cid * NS + sid
            pltpu.async_copy(src_hbm.at[wid], src_v, sem_g[0])
            pltpu.async_copy(dst_hbm.at[wid], dst_v, sem_g[0])

        for j in range(RPT // ZROWS):
            pltpu.sync_copy(z_v, acc_sh.at[pl.ds(sid * RPT + j * ZROWS,
                                                 ZROWS)])

        if p_total == 2:
            for r in range(2):
                pltpu.make_async_copy(
                    src_hbm.at[sid + r * NS],
                    src_v.at[pl.ds(r * NCHUNK, NCHUNK)], sem_g[0]).wait()
                pltpu.make_async_copy(
                    dst_hbm.at[sid + r * NS],
                    dst_v.at[pl.ds(r * NCHUNK, NCHUNK)], sem_g[0]).wait()
        else:
            pltpu.make_async_copy(src_hbm.at[0], src_v, sem_g[0]).wait()
            pltpu.make_async_copy(dst_hbm.at[0], dst_v, sem_g[0]).wait()
        plsc.subcore_barrier()

        def run_slab(x_hbm):
            def wait_gather(b):
                pltpu.make_async_copy(x_hbm.at[pl.ds(0, CHUNK)], bufs[b],
                                      sem_g[b]).wait()

            def wait_add(b, c):
                pltpu.make_async_copy(bufs[b], acc_sh.at[dst_v.at[c]],
                                      sem_a[b]).wait()

            for b in range(NBUF):
                pltpu.async_copy(x_hbm.at[src_v.at[b]], bufs[b], sem_g[b])

            nmain = (nch - 1) // NBUF * NBUF

            @pl.loop(0, nmain // NBUF)
            def _(i):
                for b in range(NBUF):
                    c = NBUF * i + b
                    wait_gather(b)
                    pltpu.async_copy(bufs[b], acc_sh.at[dst_v.at[c]],
                                     sem_a[b], add=True)
                for b in range(NBUF):
                    c = NBUF * i + b
                    cn = c + NBUF
                    wait_add(b, c)

                    @pl.when(cn < nch)
                    def _():
                        pltpu.async_copy(x_hbm.at[src_v.at[cn]], bufs[b],
                                         sem_g[b])

            for c in range(nmain, nch):
                b = c % NBUF
                wait_gather(b)
                pltpu.sync_copy(bufs[b], acc_sh.at[dst_v.at[c]], add=True)

        if p_total == 2:
            @pl.when(cid == 0)
            def _():
                run_slab(x_hbms[0])

            @pl.when(cid == 1)
            def _():
                run_slab(x_hbms[1])
        else:
            run_slab(x_hbms[0])

        plsc.subcore_barrier()

        pltpu.sync_copy(acc_sh.at[pl.ds(sid * RPT_RD, RPT_RD)],
                        out_hbm.at[cid, pl.ds(sid * RPT_RD, RPT_RD)])

        @pl.when(sid == NS - 1)
        def _():
            tail = N_NODES - NS * RPT_RD
            pltpu.sync_copy(
                acc_sh.at[pl.ds(NS * RPT_RD, tail)],
                out_hbm.at[cid, pl.ds(NS * RPT_RD, tail)])

    return kern(*parts, src, dst)


def _dense_layer(acc, xs, w_rel_t, b_rel, w_root_t, p_total, out_slabs):
    n = acc.shape[1]
    d_out = w_rel_t.shape[2]
    x_total = len(xs)
    blk = 1000

    def body(*refs):
        acc_ref = refs[0]
        x_refs = refs[1:1 + x_total]
        wr_ref, b_ref, wt_ref = refs[1 + x_total:4 + x_total]
        o_refs = refs[4 + x_total:]
        y = b_ref[...]
        if p_total == 2:
            for p in range(2):
                y = y + jnp.dot(acc_ref[p], wr_ref[p],
                                precision=lax.Precision.HIGHEST,
                                preferred_element_type=jnp.float32)
        else:
            a = acc_ref[0] + acc_ref[1]
            y = y + jnp.dot(a, wr_ref[0],
                            precision=lax.Precision.HIGHEST,
                            preferred_element_type=jnp.float32)
        for q in range(x_total):
            y = y + jnp.dot(x_refs[q][...], wt_ref[q],
                            precision=lax.Precision.HIGHEST,
                            preferred_element_type=jnp.float32)
        y = jnp.where(y >= 0, y, 0.01 * y)
        ow = d_out // out_slabs
        for s in range(out_slabs):
            o_refs[s][...] = y[:, s * ow:(s + 1) * ow]

    return pl.pallas_call(
        body,
        grid=(n // blk,),
        in_specs=[pl.BlockSpec((2, blk, DCOL), lambda i: (0, i, 0))]
        + [pl.BlockSpec((blk, DCOL), lambda i: (i, 0))] * x_total
        + [
            pl.BlockSpec((p_total, DCOL, d_out), lambda i: (0, 0, 0)),
            pl.BlockSpec((1, d_out), lambda i: (0, 0)),
            pl.BlockSpec((x_total, DCOL, d_out), lambda i: (0, 0, 0)),
        ],
        out_specs=[pl.BlockSpec((blk, d_out // out_slabs),
                                lambda i: (i, 0))] * out_slabs,
        out_shape=[jax.ShapeDtypeStruct((n, d_out // out_slabs),
                                        jnp.float32)] * out_slabs,
    )(acc, *xs, w_rel_t, b_rel, w_root_t)


def kernel(x, edge_index,
           W_rel_e0, b_rel_e0, W_root_e0,
           W_rel_e1, b_rel_e1, W_root_e1,
           W_rel_d0, b_rel_d0, W_root_d0,
           W_rel_d1, b_rel_d1, W_root_d1):
    src = edge_index[0].reshape(NW, NCHUNK, CHUNK)
    dst = edge_index[1].reshape(NW, NCHUNK, CHUNK)

    def layer(x_slabs, w_rel, b_rel, w_root, out_slabs):
        p_total = len(x_slabs)
        acc = _sc_segment_sum(x_slabs, src, dst)
        wr_t = w_rel.T.reshape(p_total, DCOL, -1)
        wt_t = w_root.T.reshape(p_total, DCOL, -1)
        outs = _dense_layer(acc, x_slabs, wr_t, b_rel.reshape(1, -1), wt_t,
                            p_total, out_slabs)
        return tuple(outs)

    x_slabs = (x[:, :DCOL], x[:, DCOL:])
    h = layer(x_slabs, W_rel_e0, b_rel_e0, W_root_e0, 2)
    emb_t = layer(h, W_rel_e1, b_rel_e1, W_root_e1, 1)
    h2 = layer(emb_t, W_rel_d0, b_rel_d0, W_root_d0, 2)
    (recon,) = layer(h2, W_rel_d1, b_rel_d1, W_root_d1, 1)
    return (recon, emb_t[0])
